# baseline (speedup 1.0000x reference)
import dataclasses
from functools import partial

import jax
import jax.numpy as jnp
from jax import lax
from jax.experimental import pallas as pl
from jax.experimental.pallas import tpu as pltpu
from jax.experimental.pallas import tpu_sc as plsc

_K = 16
_STRIDE = 8
_ROWS = 256


def _knn_body(n_extract, want_di, s_dim, keys_ref, q0_ref, *refs):
    if want_di:
        dist_ref, idx_ref, outx_ref, qcur_ref = refs
    else:
        outx_ref, qcur_ref = refs
    s = pl.program_id(s_dim)
    nkeys = keys_ref.shape[3]

    @pl.when(s == 0)
    def _():
        qcur_ref[...] = q0_ref[...].reshape(qcur_ref.shape)

    q = qcur_ref[...]
    qx, qy, qz = q[:, 0:1], q[:, 1:2], q[:, 2:3]
    kx = keys_ref[0, 0, 0:1, :]
    ky = keys_ref[0, 0, 1:2, :]
    kz = keys_ref[0, 0, 2:3, :]
    dx = qx - kx
    dy = qy - ky
    dz = qz - kz
    dd = (dx * dx + dy * dy) + dz * dz
    iota_f = lax.broadcasted_iota(jnp.int32, dd.shape, 1).astype(jnp.float32)
    big = jnp.float32(2.0 * nkeys)
    ms, mis = [], []
    for e in range(n_extract):
        m = jnp.min(dd, axis=1, keepdims=True)
        mi_f = jnp.min(jnp.where(dd == m, iota_f, big), axis=1,
                       keepdims=True)
        onehot = iota_f == mi_f
        if e == 0:
            ohf = onehot.astype(jnp.float32)
            nx = jnp.sum(ohf * kx, axis=1, keepdims=True)
            ny = jnp.sum(ohf * ky, axis=1, keepdims=True)
            nz = jnp.sum(ohf * kz, axis=1, keepdims=True)
            new_q = jnp.concatenate([nx, ny, nz], axis=1)
            qcur_ref[...] = new_q
            outx_ref[0, 0] = new_q
        if e < n_extract - 1:
            dd = jnp.where(onehot, jnp.float32(jnp.inf), dd)
        ms.append(m)
        mis.append(mi_f)
    if want_di:
        dist_ref[0, 0] = jnp.concatenate(ms, axis=1)
        idx_ref[0, 0] = jnp.concatenate(mis, axis=1).astype(jnp.int32)


def _knn_call(keys, q0, n_extract, want_di, interpret=False):
    c, t, _, n = keys.shape
    r = q0.shape[1]
    grid = (c, r // _ROWS, t)
    omap = lambda ci, ri, si: (ci, si, ri, 0)
    out_shapes = []
    out_specs = []
    if want_di:
        out_shapes += [jax.ShapeDtypeStruct((c, t, r, _K), jnp.float32),
                       jax.ShapeDtypeStruct((c, t, r, _K), jnp.int32)]
        out_specs += [pl.BlockSpec((1, 1, _ROWS, _K), omap)] * 2
    out_shapes.append(jax.ShapeDtypeStruct((c, t, r, 3), jnp.float32))
    out_specs.append(pl.BlockSpec((1, 1, _ROWS, 3), omap))
    return pl.pallas_call(
        partial(_knn_body, n_extract, want_di, 2),
        grid=grid,
        in_specs=[
            pl.BlockSpec((1, 1, 3, n), lambda ci, ri, si: (ci, si, 0, 0)),
            pl.BlockSpec((1, _ROWS, 3), lambda ci, ri, si: (ci, ri, 0)),
        ],
        out_specs=out_specs,
        out_shape=out_shapes,
        scratch_shapes=[pltpu.VMEM((_ROWS, 3), jnp.float32)],
        interpret=interpret,
    )(keys, q0)


def _knn_merged(keys, q0_pair, interpret=False):
    c, t, _, n = keys.shape
    r = q0_pair.shape[2]
    nb = r // _ROWS
    grid = (2, c, nb, t)

    def seff(di, si):
        return si + di * (t - 1 - 2 * si)

    kmap = lambda di, ci, ri, si: (ci, seff(di, si), 0, 0)
    omap = lambda di, ci, ri, si: (ci, seff(di, si), di * nb + ri, 0)
    out_shapes = [jax.ShapeDtypeStruct((c, t, 2 * r, _K), jnp.float32),
                  jax.ShapeDtypeStruct((c, t, 2 * r, _K), jnp.int32),
                  jax.ShapeDtypeStruct((c, t, 2 * r, 3), jnp.float32)]
    out_specs = [pl.BlockSpec((1, 1, _ROWS, _K), omap),
                 pl.BlockSpec((1, 1, _ROWS, _K), omap),
                 pl.BlockSpec((1, 1, _ROWS, 3), omap)]
    return pl.pallas_call(
        partial(_knn_body, _K, True, 3),
        grid=grid,
        in_specs=[
            pl.BlockSpec((1, 1, 3, n), kmap),
            pl.BlockSpec((1, 1, _ROWS, 3),
                         lambda di, ci, ri, si: (di, ci, ri, 0)),
        ],
        out_specs=out_specs,
        out_shape=out_shapes,
        scratch_shapes=[pltpu.VMEM((_ROWS, 3), jnp.float32)],
        interpret=interpret,
    )(keys, q0_pair)


def _sc_gather(tab, idx):
    g_tot, m = idx.shape
    npts8 = tab.shape[1] // 2
    mh = m // 2
    nw = 32
    gpw = g_tot // nw
    mesh = plsc.VectorSubcoreMesh(core_axis_name="c", subcore_axis_name="s")
    cp = pltpu.CompilerParams()
    if "needs_layout_passes" in pltpu.CompilerParams.__dataclass_fields__:
        cp = dataclasses.replace(cp, needs_layout_passes=False)

    @partial(pl.kernel,
             out_type=[jax.ShapeDtypeStruct((g_tot, 3 * m), jnp.float32),
                       jax.ShapeDtypeStruct((g_tot, 3 * m), jnp.float32)],
             mesh=mesh,
             scratch_types=[pltpu.VMEM((tab.shape[1],), jnp.float32),
                            pltpu.VMEM((mh,), jnp.int32),
                            pltpu.VMEM((3 * mh,), jnp.float32),
                            pltpu.VMEM((3 * mh,), jnp.float32)],
             compiler_params=cp)
    def k(tab_hbm, idx_hbm, pp_hbm, pf_hbm, tab_v, idx_v, pp_v, pf_v):
        wid = lax.axis_index("s") * 2 + lax.axis_index("c")
        lane = lax.iota(jnp.int32, 16)

        @pl.loop(0, gpw)
        def _(gi):
            g = wid * gpw + gi
            pltpu.sync_copy(tab_hbm.at[g], tab_v)
            for hc in range(2):
                pltpu.sync_copy(idx_hbm.at[g, pl.ds(hc * mh, mh)], idx_v)
                off = hc * npts8

                @pl.loop(0, mh // 16)
                def _(i):
                    iv = idx_v[pl.ds(i * 16, 16)] * 8 + off
                    pos = lane * 3 + i * 48
                    for c in range(3):
                        plsc.store_scatter(pp_v, [pos + c],
                                           plsc.load_gather(tab_v, [iv + c]))
                        plsc.store_scatter(
                            pf_v, [pos + c],
                            plsc.load_gather(tab_v, [iv + (c + 3)]))

                pltpu.sync_copy(pp_v, pp_hbm.at[g, pl.ds(hc * 3 * mh, 3 * mh)])
                pltpu.sync_copy(pf_v, pf_hbm.at[g, pl.ds(hc * 3 * mh, 3 * mh)])

    return k(tab, idx)


def kernel(point_seq):
    b, t, n, d = point_seq.shape
    nseg = t // _STRIDE
    nc = nseg * b
    half = n // 2
    perm = jax.random.permutation(jax.random.key(42), n)
    perm_a, perm_b = perm[:half], perm[half:]
    inv_perm = jnp.argsort(perm)

    ps_t = point_seq.transpose(0, 1, 3, 2)
    psr_t = ps_t.reshape(b, nseg, _STRIDE, 3, n)
    keys_f = psr_t.reshape(nc, _STRIDE, 3, n)

    psr = point_seq.reshape(b, nseg, _STRIDE, n, d)
    first = psr[:, :, 0]
    last = psr[:, :, -1]
    q0_fa = first[:, :, perm_a].reshape(nc, half, 3)
    q0_fb = first[:, :, perm_b].reshape(nc, half, 3)
    q0_bw = last[:, :, perm_a].reshape(nc, half, 3)

    dist_m, idx_m, ox_m = _knn_merged(keys_f, jnp.stack([q0_fa, q0_bw]))
    (ox_b,) = _knn_call(keys_f, q0_fb, 1, False)

    def _seq(x):
        return x.reshape((b, nseg) + x.shape[1:]).reshape((b, t) + x.shape[2:])

    distances = _seq(dist_m)
    idxs = _seq(idx_m)
    ox = jnp.concatenate([_seq(ox_m)[:, :, :half], _seq(ox_b)], axis=2)
    out_x = jnp.take(ox, inv_perm, axis=2)

    src0 = jnp.concatenate([psr[:, :, :1], psr[:, :, :-1]], axis=2)
    src1 = jnp.concatenate([psr[:, :, 1:], psr[:, :, -1:]], axis=2)
    srcs = jnp.stack([src0.reshape(b, t, n, d),
                      src1.reshape(b, t, n, d)], axis=2)
    pts = jnp.broadcast_to(point_seq[:, :, None], (b, t, 2, n, d))
    tab = jnp.pad(jnp.concatenate([pts, srcs], -1),
                  ((0, 0),) * 4 + ((0, 8 - 2 * d),))
    pp, pf = _sc_gather(tab.reshape(b * t, 2 * n * 8),
                        idxs.reshape(b * t, n * _K))
    patchlet_points = pp.reshape(b, t, n, _K, d)
    patchlet_feats = pf.reshape(b, t, n, _K, d)
    return patchlet_points, patchlet_feats, distances, idxs, out_x


# speedup vs baseline: 1.0222x; 1.0222x over previous
import dataclasses
from functools import partial

import jax
import jax.numpy as jnp
from jax import lax
from jax.experimental import pallas as pl
from jax.experimental.pallas import tpu as pltpu
from jax.experimental.pallas import tpu_sc as plsc

_K = 16
_STRIDE = 8
_ROWS = 256


def _knn_body(n_extract, want_di, s_dim, keys_ref, q0_ref, *refs):
    if want_di:
        dist_ref, idx_ref, outx_ref, qcur_ref = refs
    else:
        outx_ref, qcur_ref = refs
    s = pl.program_id(s_dim)
    nkeys = keys_ref.shape[3]

    @pl.when(s == 0)
    def _():
        qcur_ref[...] = q0_ref[...].reshape(qcur_ref.shape)

    q = qcur_ref[...]
    qx, qy, qz = q[:, 0:1], q[:, 1:2], q[:, 2:3]
    kx = keys_ref[0, 0, 0:1, :]
    ky = keys_ref[0, 0, 1:2, :]
    kz = keys_ref[0, 0, 2:3, :]
    dx = qx - kx
    dy = qy - ky
    dz = qz - kz
    dd = (dx * dx + dy * dy) + dz * dz
    iota_f = lax.broadcasted_iota(jnp.int32, dd.shape, 1).astype(jnp.float32)
    big = jnp.float32(2.0 * nkeys)
    ms, mis = [], []
    for e in range(n_extract):
        m = jnp.min(dd, axis=1, keepdims=True)
        mi_f = jnp.min(jnp.where(dd == m, iota_f, big), axis=1,
                       keepdims=True)
        onehot = iota_f == mi_f
        if e == 0:
            ohf = onehot.astype(jnp.float32)
            nx = jnp.sum(ohf * kx, axis=1, keepdims=True)
            ny = jnp.sum(ohf * ky, axis=1, keepdims=True)
            nz = jnp.sum(ohf * kz, axis=1, keepdims=True)
            new_q = jnp.concatenate([nx, ny, nz], axis=1)
            qcur_ref[...] = new_q
            outx_ref[0, 0] = new_q
        if e < n_extract - 1:
            dd = jnp.where(onehot, jnp.float32(jnp.inf), dd)
        ms.append(m)
        mis.append(mi_f)
    if want_di:
        dist_ref[0, 0] = jnp.concatenate(ms, axis=1)
        idx_ref[0, 0] = jnp.concatenate(mis, axis=1).astype(jnp.int32)


def _knn_call(keys, q0, n_extract, want_di, flip_t=False, interpret=False):
    c, t, _, n = keys.shape
    r = q0.shape[1]
    grid = (c, r // _ROWS, t)
    if flip_t:
        omap = lambda ci, ri, si: (ci, t - 1 - si, ri, 0)
    else:
        omap = lambda ci, ri, si: (ci, si, ri, 0)
    out_shapes = []
    out_specs = []
    if want_di:
        out_shapes += [jax.ShapeDtypeStruct((c, t, r, _K), jnp.float32),
                       jax.ShapeDtypeStruct((c, t, r, _K), jnp.int32)]
        out_specs += [pl.BlockSpec((1, 1, _ROWS, _K), omap)] * 2
    out_shapes.append(jax.ShapeDtypeStruct((c, t, r, 3), jnp.float32))
    out_specs.append(pl.BlockSpec((1, 1, _ROWS, 3), omap))
    return pl.pallas_call(
        partial(_knn_body, n_extract, want_di, 2),
        grid=grid,
        in_specs=[
            pl.BlockSpec((1, 1, 3, n), lambda ci, ri, si: (ci, si, 0, 0)),
            pl.BlockSpec((1, _ROWS, 3), lambda ci, ri, si: (ci, ri, 0)),
        ],
        out_specs=out_specs,
        out_shape=out_shapes,
        scratch_shapes=[pltpu.VMEM((_ROWS, 3), jnp.float32)],
        interpret=interpret,
    )(keys, q0)


def _sc_gather(tab, idx):
    g_tot, m = idx.shape
    nw = 32
    gpw = g_tot // nw
    mesh = plsc.VectorSubcoreMesh(core_axis_name="c", subcore_axis_name="s")
    cp = pltpu.CompilerParams()
    if "needs_layout_passes" in pltpu.CompilerParams.__dataclass_fields__:
        cp = dataclasses.replace(cp, needs_layout_passes=False)

    @partial(pl.kernel,
             out_type=[jax.ShapeDtypeStruct((g_tot, 3 * m), jnp.float32),
                       jax.ShapeDtypeStruct((g_tot, 3 * m), jnp.float32)],
             mesh=mesh,
             scratch_types=[pltpu.VMEM((tab.shape[1],), jnp.float32),
                            pltpu.VMEM((m,), jnp.int32),
                            pltpu.VMEM((3 * m,), jnp.float32),
                            pltpu.VMEM((3 * m,), jnp.float32)],
             compiler_params=cp)
    def k(tab_hbm, idx_hbm, pp_hbm, pf_hbm, tab_v, idx_v, pp_v, pf_v):
        wid = lax.axis_index("s") * 2 + lax.axis_index("c")
        lane = lax.iota(jnp.int32, 16)

        @pl.loop(0, gpw)
        def _(gi):
            g = wid * gpw + gi
            pltpu.sync_copy(tab_hbm.at[g], tab_v)
            pltpu.sync_copy(idx_hbm.at[g], idx_v)

            @pl.loop(0, m // 16)
            def _(i):
                iv = idx_v[pl.ds(i * 16, 16)] * 8
                pos = lane * 3 + i * 48
                for c in range(3):
                    plsc.store_scatter(pp_v, [pos + c],
                                       plsc.load_gather(tab_v, [iv + c]))
                    plsc.store_scatter(pf_v, [pos + c],
                                       plsc.load_gather(tab_v, [iv + (c + 3)]))

            pltpu.sync_copy(pp_v, pp_hbm.at[g])
            pltpu.sync_copy(pf_v, pf_hbm.at[g])

    return k(tab, idx)


def kernel(point_seq):
    b, t, n, d = point_seq.shape
    nseg = t // _STRIDE
    nc = nseg * b
    half = n // 2
    perm = jax.random.permutation(jax.random.key(42), n)
    perm_a, perm_b = perm[:half], perm[half:]
    inv_perm = jnp.argsort(perm)

    ps_t = point_seq.transpose(0, 1, 3, 2)
    psr_t = ps_t.reshape(b, nseg, _STRIDE, 3, n)
    keys_f = psr_t.reshape(nc, _STRIDE, 3, n)
    keys_r = psr_t[:, :, ::-1].reshape(nc, _STRIDE, 3, n)

    psr = point_seq.reshape(b, nseg, _STRIDE, n, d)
    first = psr[:, :, 0]
    last = psr[:, :, -1]
    q0_fa = first[:, :, perm_a].reshape(nc, half, 3)
    q0_fb = first[:, :, perm_b].reshape(nc, half, 3)
    q0_bw = last[:, :, perm_a].reshape(nc, half, 3)

    dist_a, idx_a, ox_a = _knn_call(keys_f, q0_fa, _K, True)
    (ox_b,) = _knn_call(keys_f, q0_fb, 1, False)
    dist_w, idx_w, _ = _knn_call(keys_r, q0_bw, _K, True, flip_t=True)

    def _seq(x):
        return x.reshape((b, nseg) + x.shape[1:]).reshape((b, t) + x.shape[2:])

    distances = jnp.concatenate([_seq(dist_a), _seq(dist_w)], axis=2)
    idxs = jnp.concatenate([_seq(idx_a), _seq(idx_w)], axis=2)
    ox = jnp.concatenate([_seq(ox_a), _seq(ox_b)], axis=2)
    out_x = jnp.take(ox, inv_perm, axis=2)

    src0 = jnp.concatenate([psr[:, :, :1], psr[:, :, :-1]], axis=2)
    src1 = jnp.concatenate([psr[:, :, 1:], psr[:, :, -1:]], axis=2)
    src0 = src0.reshape(b, t, n, d)
    src1 = src1.reshape(b, t, n, d)
    pad = ((0, 0),) * 3 + ((0, 8 - 2 * d),)
    tab0 = jnp.pad(jnp.concatenate([point_seq, src0], -1), pad)
    tab1 = jnp.pad(jnp.concatenate([point_seq, src1], -1), pad)
    m = half * _K
    pp0, pf0 = _sc_gather(tab0.reshape(b * t, n * 8),
                          _seq(idx_a).reshape(b * t, m))
    pp1, pf1 = _sc_gather(tab1.reshape(b * t, n * 8),
                          _seq(idx_w).reshape(b * t, m))

    def _halves(x0, x1):
        return jnp.concatenate([x0.reshape(b, t, half, _K, d),
                                x1.reshape(b, t, half, _K, d)], axis=2)

    patchlet_points = _halves(pp0, pp1)
    patchlet_feats = _halves(pf0, pf1)
    return patchlet_points, patchlet_feats, distances, idxs, out_x
